# baseline (speedup 1.0000x reference)
import functools

import jax
import jax.numpy as jnp
from jax import lax
from jax.experimental import pallas as pl
from jax.experimental.pallas import tpu as pltpu


def _upconv_body(x_ref, w_ref, b_ref, o_ref):
    acc = jnp.dot(x_ref[...], w_ref[...], preferred_element_type=jnp.float32)
    o_ref[...] = (acc + b_ref[...]).astype(jnp.bfloat16)


def _upsample_2x2(x_nhwc, w_up, b_up, *, tile_m=2048):
    N, H, W, Cin = x_nhwc.shape
    Chalf = w_up.shape[1]
    w = jnp.transpose(w_up, (0, 2, 3, 1)).reshape(Cin, 4 * Chalf).astype(jnp.bfloat16)
    b = jnp.tile(b_up, 4).reshape(1, 4 * Chalf)

    M = N * H * W
    tm = min(tile_m, M)
    xf = x_nhwc.reshape(M, Cin)

    out = pl.pallas_call(
        _upconv_body,
        out_shape=jax.ShapeDtypeStruct((M, 4 * Chalf), jnp.bfloat16),
        grid=(pl.cdiv(M, tm),),
        in_specs=[
            pl.BlockSpec((tm, Cin), lambda i: (i, 0)),
            pl.BlockSpec((Cin, 4 * Chalf), lambda i: (0, 0)),
            pl.BlockSpec((1, 4 * Chalf), lambda i: (0, 0)),
        ],
        out_specs=pl.BlockSpec((tm, 4 * Chalf), lambda i: (i, 0)),
        compiler_params=pltpu.CompilerParams(dimension_semantics=("parallel",)),
    )(xf, w, b)

    out = out.reshape(N, H, W, 2, 2, Chalf)
    out = jnp.transpose(out, (0, 1, 3, 2, 4, 5)).reshape(N, 2 * H, 2 * W, Chalf)
    return out


def _strip_body(
    up_hbm, x2_hbm, w1_ref, b1_ref, w2_ref, b2_ref,
    o_ref,
    up_buf, x2_buf, dma_sem, h1_ref,
    *, TH, OH, OW, Chalf, Cout,
):
    n = pl.program_id(0)
    s = pl.program_id(1)
    r0 = s * TH
    row0 = n * (OH + 4) + r0

    cp_up = pltpu.make_async_copy(up_hbm.at[pl.ds(row0, TH + 4)], up_buf, dma_sem.at[0])
    cp_x2 = pltpu.make_async_copy(x2_hbm.at[pl.ds(row0, TH + 4)], x2_buf, dma_sem.at[1])
    cp_up.start()
    cp_x2.start()
    cp_up.wait()
    cp_x2.wait()

    Hr, Wr = TH + 2, OW + 2

    taps = []
    for dy in range(3):
        for dx in range(3):
            taps.append(up_buf[dy:dy + Hr, dx:dx + Wr, :].reshape(Hr * Wr, Chalf))
            taps.append(x2_buf[dy:dy + Hr, dx:dx + Wr, :].reshape(Hr * Wr, Chalf))
    patches = jnp.concatenate(taps, axis=-1)
    h1 = jnp.dot(patches, w1_ref[...], preferred_element_type=jnp.float32) + b1_ref[...]
    h1 = jnp.maximum(h1, 0.0).reshape(Hr, Wr, Cout)

    r = lax.broadcasted_iota(jnp.int32, (Hr, Wr, Cout), 0) + (r0 - 1)
    c = lax.broadcasted_iota(jnp.int32, (Hr, Wr, Cout), 1) - 1
    valid = (r >= 0) & (r < OH) & (c >= 0) & (c < OW)
    h1_ref[...] = jnp.where(valid, h1, 0.0).astype(jnp.bfloat16)

    taps2 = []
    for dy in range(3):
        for dx in range(3):
            taps2.append(h1_ref[dy:dy + TH, dx:dx + OW, :].reshape(TH * OW, Cout))
    p2 = jnp.concatenate(taps2, axis=-1)
    h2 = jnp.dot(p2, w2_ref[...], preferred_element_type=jnp.float32) + b2_ref[...]
    o_ref[...] = jnp.maximum(h2, 0.0).reshape(TH, OW, Cout)


def _fused_double_conv(up, x2, w1, b1, w2, b2, *, strip_rows=16):
    N, OH, OW, Chalf = up.shape
    Cin = 2 * Chalf
    Cout = w1.shape[0]

    th = min(strip_rows, OH)
    while OH % th:
        th -= 1
    n_strips = OH // th

    up_p = jnp.pad(up, ((0, 0), (2, 2), (2, 2), (0, 0))).reshape(N * (OH + 4), OW + 4, Chalf)
    x2_p = jnp.pad(x2, ((0, 0), (2, 2), (2, 2), (0, 0))).reshape(N * (OH + 4), OW + 4, Chalf)

    w1_mat = jnp.transpose(w1, (2, 3, 1, 0)).reshape(9 * Cin, Cout).astype(jnp.bfloat16)
    w2_mat = jnp.transpose(w2, (2, 3, 1, 0)).reshape(9 * Cout, Cout).astype(jnp.bfloat16)
    b1_row = b1.reshape(1, Cout)
    b2_row = b2.reshape(1, Cout)

    body = functools.partial(
        _strip_body, TH=th, OH=OH, OW=OW, Chalf=Chalf, Cout=Cout)

    out = pl.pallas_call(
        body,
        out_shape=jax.ShapeDtypeStruct((N * OH, OW, Cout), jnp.float32),
        grid=(N, n_strips),
        in_specs=[
            pl.BlockSpec(memory_space=pl.ANY),
            pl.BlockSpec(memory_space=pl.ANY),
            pl.BlockSpec((9 * Cin, Cout), lambda n, s: (0, 0)),
            pl.BlockSpec((1, Cout), lambda n, s: (0, 0)),
            pl.BlockSpec((9 * Cout, Cout), lambda n, s: (0, 0)),
            pl.BlockSpec((1, Cout), lambda n, s: (0, 0)),
        ],
        out_specs=pl.BlockSpec((th, OW, Cout), lambda n, s: (n * n_strips + s, 0, 0)),
        scratch_shapes=[
            pltpu.VMEM((th + 4, OW + 4, Chalf), jnp.bfloat16),
            pltpu.VMEM((th + 4, OW + 4, Chalf), jnp.bfloat16),
            pltpu.SemaphoreType.DMA((2,)),
            pltpu.VMEM((th + 2, OW + 2, Cout), jnp.bfloat16),
        ],
        compiler_params=pltpu.CompilerParams(
            dimension_semantics=("parallel", "parallel")),
    )(up_p, x2_p, w1_mat, b1_row, w2_mat, b2_row)

    return out.reshape(N, OH, OW, Cout)


def kernel(x1, x2, w_up, b_up, w1, b1, w2, b2):
    x1 = jnp.transpose(x1, (0, 2, 3, 1)).astype(jnp.bfloat16)
    x2 = jnp.transpose(x2, (0, 2, 3, 1)).astype(jnp.bfloat16)
    up = _upsample_2x2(x1, w_up, b_up)
    out = _fused_double_conv(up, x2, w1, b1, w2, b2, strip_rows=16)
    return jnp.transpose(out, (0, 3, 1, 2))


# speedup vs baseline: 1.1374x; 1.1374x over previous
import functools

import jax
import jax.numpy as jnp
from jax import lax
from jax.experimental import pallas as pl
from jax.experimental.pallas import tpu as pltpu


def _rolls(arr, w):
    col = lax.broadcasted_iota(jnp.int32, arr.shape, 1) % w
    rot_l = jnp.concatenate([arr[:, -1:], arr[:, :-1]], axis=1)
    rot_r = jnp.concatenate([arr[:, 1:], arr[:, :1]], axis=1)
    left = jnp.where(col == 0, 0.0, rot_l)
    right = jnp.where(col == w - 1, 0.0, rot_r)
    return (left, arr, right)


def _fused_body(
    x1_hbm, x2_hbm, wup_ref, bup_ref, w1_ref, b1_ref, w2_ref, b2_ref,
    o_ref,
    xw, x2b, upf, p1, h1f, p2, sem,
    *, TH, H, W, OH, OW, Chalf, Cin, Cout,
):
    n = pl.program_id(0)
    s = pl.program_id(1)
    r0 = s * TH
    CW = TH // 2 + 4
    Hr = TH + 2

    cp1 = pltpu.make_async_copy(
        x1_hbm.at[n, :, pl.ds((TH // 2) * s * W, CW * W)], xw, sem.at[0])
    cp2 = pltpu.make_async_copy(
        x2_hbm.at[n, :, pl.ds(r0 * OW, (TH + 4) * OW)], x2b, sem.at[1])
    cp1.start()
    cp2.start()
    cp1.wait()
    cp2.wait()

    um = jnp.dot(wup_ref[...], xw[...], preferred_element_type=jnp.float32)
    row = lax.broadcasted_iota(jnp.int32, um.shape, 0)
    lane = lax.broadcasted_iota(jnp.int32, um.shape, 1)
    ki = row // (2 * Chalf)
    iloc = lane // W
    u_abs = 2 * (TH // 2) * s - 4 + 2 * iloc + ki
    um = jnp.where((u_abs >= 0) & (u_abs < OH), um + bup_ref[...], 0.0)
    u5 = um.reshape(2, 2, Chalf, CW, W).astype(jnp.bfloat16)
    upf[...] = jnp.transpose(u5, (2, 3, 0, 4, 1)).reshape(Chalf, 2 * CW * OW)

    up_taps = _rolls(upf[...], OW)
    x2_taps = _rolls(x2b[...], OW)
    for dy in range(3):
        for dx in range(3):
            k0 = (dy * 3 + dx) * Cin
            p1[k0:k0 + Chalf, :] = up_taps[dx][:, (2 + dy) * OW:(2 + dy + Hr) * OW]
            p1[k0 + Chalf:k0 + Cin, :] = x2_taps[dx][:, dy * OW:(dy + Hr) * OW]
    h1 = jnp.dot(w1_ref[...], p1[...], preferred_element_type=jnp.float32)
    h1 = jnp.maximum(h1 + b1_ref[...], 0.0)
    yy = lax.broadcasted_iota(jnp.int32, h1.shape, 1) // OW + (r0 - 1)
    h1f[...] = jnp.where((yy >= 0) & (yy < OH), h1, 0.0).astype(jnp.bfloat16)

    h1_taps = _rolls(h1f[...], OW)
    for dy in range(3):
        for dx in range(3):
            k0 = (dy * 3 + dx) * Cout
            p2[k0:k0 + Cout, :] = h1_taps[dx][:, dy * OW:(dy + TH) * OW]
    h2 = jnp.dot(w2_ref[...], p2[...], preferred_element_type=jnp.float32)
    o_ref[...] = jnp.maximum(h2 + b2_ref[...], 0.0)[None]


def kernel(x1, x2, w_up, b_up, w1, b1, w2, b2):
    N, Cin, H, W = x1.shape
    Chalf = Cin // 2
    OH, OW = 2 * H, 2 * W
    Cout = w1.shape[0]
    TH = 16
    n_strips = OH // TH
    CW = TH // 2 + 4

    x1p = jnp.pad(x1.astype(jnp.bfloat16), ((0, 0), (0, 0), (2, 2), (0, 0))
                  ).reshape(N, Cin, (H + 4) * W)
    x2p = jnp.pad(x2.astype(jnp.bfloat16), ((0, 0), (0, 0), (2, 2), (0, 0))
                  ).reshape(N, Chalf, (OH + 4) * OW)

    wupT = jnp.transpose(w_up, (2, 3, 1, 0)).reshape(4 * Chalf, Cin).astype(jnp.bfloat16)
    w1t = jnp.transpose(w1, (0, 2, 3, 1)).reshape(Cout, 9 * Cin).astype(jnp.bfloat16)
    w2t = jnp.transpose(w2, (0, 2, 3, 1)).reshape(Cout, 9 * Cout).astype(jnp.bfloat16)
    bup4 = jnp.tile(b_up, 4).reshape(4 * Chalf, 1)
    b1c = b1.reshape(Cout, 1)
    b2c = b2.reshape(Cout, 1)

    body = functools.partial(
        _fused_body, TH=TH, H=H, W=W, OH=OH, OW=OW,
        Chalf=Chalf, Cin=Cin, Cout=Cout)

    out = pl.pallas_call(
        body,
        out_shape=jax.ShapeDtypeStruct((N, Cout, OH * OW), jnp.float32),
        grid=(N, n_strips),
        in_specs=[
            pl.BlockSpec(memory_space=pl.ANY),
            pl.BlockSpec(memory_space=pl.ANY),
            pl.BlockSpec((4 * Chalf, Cin), lambda n, s: (0, 0)),
            pl.BlockSpec((4 * Chalf, 1), lambda n, s: (0, 0)),
            pl.BlockSpec((Cout, 9 * Cin), lambda n, s: (0, 0)),
            pl.BlockSpec((Cout, 1), lambda n, s: (0, 0)),
            pl.BlockSpec((Cout, 9 * Cout), lambda n, s: (0, 0)),
            pl.BlockSpec((Cout, 1), lambda n, s: (0, 0)),
        ],
        out_specs=pl.BlockSpec((1, Cout, TH * OW), lambda n, s: (n, 0, s)),
        scratch_shapes=[
            pltpu.VMEM((Cin, CW * W), jnp.bfloat16),
            pltpu.VMEM((Chalf, (TH + 4) * OW), jnp.bfloat16),
            pltpu.VMEM((Chalf, (2 * CW) * OW), jnp.bfloat16),
            pltpu.VMEM((9 * Cin, (TH + 2) * OW), jnp.bfloat16),
            pltpu.VMEM((Cout, (TH + 2) * OW), jnp.bfloat16),
            pltpu.VMEM((9 * Cout, TH * OW), jnp.bfloat16),
            pltpu.SemaphoreType.DMA((2,)),
        ],
        compiler_params=pltpu.CompilerParams(
            dimension_semantics=("parallel", "parallel")),
    )(x1p, x2p, wupT, bup4, w1t, b1c, w2t, b2c)

    return out.reshape(N, Cout, OH, OW)


# speedup vs baseline: 3.9183x; 3.4451x over previous
import functools

import jax
import jax.numpy as jnp
from jax import lax
from jax.experimental import pallas as pl
from jax.experimental.pallas import tpu as pltpu


def _rolls(arr, w):
    col = lax.broadcasted_iota(jnp.int32, arr.shape, 1) % w
    rot_l = jnp.concatenate([arr[:, -1:], arr[:, :-1]], axis=1)
    rot_r = jnp.concatenate([arr[:, 1:], arr[:, :1]], axis=1)
    left = jnp.where(col == 0, 0.0, rot_l)
    right = jnp.where(col == w - 1, 0.0, rot_r)
    return (left, arr, right)


def _fused_body(
    x1_hbm, x2_hbm, wup_ref, bup_ref, w1_ref, b1_ref, w2_ref, b2_ref,
    o_ref,
    xw, x2b, upf, p1, h1f, p2, sem,
    *, TH, H, W, OH, OW, Chalf, Cin, Cout,
):
    n = pl.program_id(0)
    s = pl.program_id(1)
    r0 = s * TH
    CW = TH // 2 + 4
    Hr = TH + 2

    cp1 = pltpu.make_async_copy(
        x1_hbm.at[n, :, pl.ds((TH // 2) * s * OW, CW * OW)], xw, sem.at[0])
    cp2 = pltpu.make_async_copy(
        x2_hbm.at[n, :, pl.ds(r0 * OW, (TH + 4) * OW)], x2b, sem.at[1])
    cp1.start()
    cp2.start()
    cp1.wait()
    cp2.wait()

    um = jnp.dot(wup_ref[...], xw[...], preferred_element_type=jnp.float32)
    lane = lax.broadcasted_iota(jnp.int32, (Chalf, CW * OW), 1)
    v_even = (lane % 2) == 0
    ev = jnp.where(v_even, um[0:Chalf], um[Chalf:2 * Chalf])
    od = jnp.where(v_even, um[2 * Chalf:3 * Chalf], um[3 * Chalf:4 * Chalf])
    u_ev = 2 * (TH // 2) * s - 4 + 2 * (lane // OW)
    bup = bup_ref[...]
    ev = jnp.where((u_ev >= 0) & (u_ev < OH), ev + bup, 0.0).astype(jnp.bfloat16)
    od = jnp.where((u_ev + 1 >= 0) & (u_ev + 1 < OH), od + bup, 0.0).astype(jnp.bfloat16)
    for i in range(CW):
        upf[:, (2 * i) * OW:(2 * i + 1) * OW] = ev[:, i * OW:(i + 1) * OW]
        upf[:, (2 * i + 1) * OW:(2 * i + 2) * OW] = od[:, i * OW:(i + 1) * OW]

    up_taps = _rolls(upf[...], OW)
    x2_taps = _rolls(x2b[...], OW)
    for dy in range(3):
        for dx in range(3):
            k0 = (dy * 3 + dx) * Cin
            p1[k0:k0 + Chalf, :] = up_taps[dx][:, (2 + dy) * OW:(2 + dy + Hr) * OW]
            p1[k0 + Chalf:k0 + Cin, :] = x2_taps[dx][:, dy * OW:(dy + Hr) * OW]
    h1 = jnp.dot(w1_ref[...], p1[...], preferred_element_type=jnp.float32)
    h1 = jnp.maximum(h1 + b1_ref[...], 0.0)
    yy = lax.broadcasted_iota(jnp.int32, h1.shape, 1) // OW + (r0 - 1)
    h1f[...] = jnp.where((yy >= 0) & (yy < OH), h1, 0.0).astype(jnp.bfloat16)

    h1_taps = _rolls(h1f[...], OW)
    for dy in range(3):
        for dx in range(3):
            k0 = (dy * 3 + dx) * Cout
            p2[k0:k0 + Cout, :] = h1_taps[dx][:, dy * OW:(dy + TH) * OW]
    h2 = jnp.dot(w2_ref[...], p2[...], preferred_element_type=jnp.float32)
    o_ref[...] = jnp.maximum(h2 + b2_ref[...], 0.0)[None]


def kernel(x1, x2, w_up, b_up, w1, b1, w2, b2):
    N, Cin, H, W = x1.shape
    Chalf = Cin // 2
    OH, OW = 2 * H, 2 * W
    Cout = w1.shape[0]
    TH = 16
    n_strips = OH // TH
    CW = TH // 2 + 4

    x1p = jnp.repeat(
        jnp.pad(x1.astype(jnp.bfloat16), ((0, 0), (0, 0), (2, 2), (0, 0))),
        2, axis=-1).reshape(N, Cin, (H + 4) * OW)
    x2p = jnp.pad(x2.astype(jnp.bfloat16), ((0, 0), (0, 0), (2, 2), (0, 0))
                  ).reshape(N, Chalf, (OH + 4) * OW)

    wupT = jnp.transpose(w_up, (2, 3, 1, 0)).reshape(4 * Chalf, Cin).astype(jnp.bfloat16)
    w1t = jnp.transpose(w1, (0, 2, 3, 1)).reshape(Cout, 9 * Cin).astype(jnp.bfloat16)
    w2t = jnp.transpose(w2, (0, 2, 3, 1)).reshape(Cout, 9 * Cout).astype(jnp.bfloat16)
    bupc = b_up.reshape(Chalf, 1)
    b1c = b1.reshape(Cout, 1)
    b2c = b2.reshape(Cout, 1)

    body = functools.partial(
        _fused_body, TH=TH, H=H, W=W, OH=OH, OW=OW,
        Chalf=Chalf, Cin=Cin, Cout=Cout)

    out = pl.pallas_call(
        body,
        out_shape=jax.ShapeDtypeStruct((N, Cout, OH * OW), jnp.float32),
        grid=(N, n_strips),
        in_specs=[
            pl.BlockSpec(memory_space=pl.ANY),
            pl.BlockSpec(memory_space=pl.ANY),
            pl.BlockSpec((4 * Chalf, Cin), lambda n, s: (0, 0)),
            pl.BlockSpec((Chalf, 1), lambda n, s: (0, 0)),
            pl.BlockSpec((Cout, 9 * Cin), lambda n, s: (0, 0)),
            pl.BlockSpec((Cout, 1), lambda n, s: (0, 0)),
            pl.BlockSpec((Cout, 9 * Cout), lambda n, s: (0, 0)),
            pl.BlockSpec((Cout, 1), lambda n, s: (0, 0)),
        ],
        out_specs=pl.BlockSpec((1, Cout, TH * OW), lambda n, s: (n, 0, s)),
        scratch_shapes=[
            pltpu.VMEM((Cin, CW * OW), jnp.bfloat16),
            pltpu.VMEM((Chalf, (TH + 4) * OW), jnp.bfloat16),
            pltpu.VMEM((Chalf, (2 * CW) * OW), jnp.bfloat16),
            pltpu.VMEM((9 * Cin, (TH + 2) * OW), jnp.bfloat16),
            pltpu.VMEM((Cout, (TH + 2) * OW), jnp.bfloat16),
            pltpu.VMEM((9 * Cout, TH * OW), jnp.bfloat16),
            pltpu.SemaphoreType.DMA((2,)),
        ],
        compiler_params=pltpu.CompilerParams(
            dimension_semantics=("parallel", "parallel")),
    )(x1p, x2p, wupT, bupc, w1t, b1c, w2t, b2c)

    return out.reshape(N, Cout, OH, OW)


# speedup vs baseline: 4.4997x; 1.1484x over previous
import functools

import jax
import jax.numpy as jnp
from jax import lax
from jax.experimental import pallas as pl
from jax.experimental.pallas import tpu as pltpu


def _rolls(arr, w):
    col = lax.broadcasted_iota(jnp.int32, arr.shape, 1) % w
    rot_l = jnp.concatenate([arr[:, -1:], arr[:, :-1]], axis=1)
    rot_r = jnp.concatenate([arr[:, 1:], arr[:, :1]], axis=1)
    left = jnp.where(col == 0, 0.0, rot_l)
    right = jnp.where(col == w - 1, 0.0, rot_r)
    return (left, arr, right)


def _fused_body(
    x1_hbm, x2_hbm, wup_ref, bup_ref, w1_ref, b1_ref, w2_ref, b2_ref,
    o_ref,
    xw, x2b, upf, p1, h1f, p2, sem,
    *, TH, H, W, OH, OW, Chalf, Cin, Cout,
):
    n = pl.program_id(0)
    s = pl.program_id(1)
    r0 = s * TH
    CW = TH // 2 + 4
    Hr = TH + 2

    cp1 = pltpu.make_async_copy(
        x1_hbm.at[n, :, pl.ds((TH // 2) * s * OW, CW * OW)], xw, sem.at[0])
    cp2 = pltpu.make_async_copy(
        x2_hbm.at[n, :, pl.ds(r0 * OW, (TH + 4) * OW)], x2b, sem.at[1])
    cp1.start()
    cp2.start()
    cp1.wait()
    cp2.wait()

    um = jnp.dot(wup_ref[...], xw[...], preferred_element_type=jnp.float32)
    lane = lax.broadcasted_iota(jnp.int32, (Chalf, CW * OW), 1)
    v_even = (lane % 2) == 0
    ev = jnp.where(v_even, um[0:Chalf], um[Chalf:2 * Chalf])
    od = jnp.where(v_even, um[2 * Chalf:3 * Chalf], um[3 * Chalf:4 * Chalf])
    u_ev = 2 * (TH // 2) * s - 4 + 2 * (lane // OW)
    bup = bup_ref[...]
    ev = jnp.where((u_ev >= 0) & (u_ev < OH), ev + bup, 0.0).astype(jnp.bfloat16)
    od = jnp.where((u_ev + 1 >= 0) & (u_ev + 1 < OH), od + bup, 0.0).astype(jnp.bfloat16)
    for i in range(CW):
        upf[:, (2 * i) * OW:(2 * i + 1) * OW] = ev[:, i * OW:(i + 1) * OW]
        upf[:, (2 * i + 1) * OW:(2 * i + 2) * OW] = od[:, i * OW:(i + 1) * OW]

    up_taps = _rolls(upf[...], OW)
    x2_taps = _rolls(x2b[...], OW)
    for dy in range(3):
        for dx in range(3):
            k0 = (dy * 3 + dx) * Cin
            p1[k0:k0 + Chalf, :] = up_taps[dx][:, (2 + dy) * OW:(2 + dy + Hr) * OW]
            p1[k0 + Chalf:k0 + Cin, :] = x2_taps[dx][:, dy * OW:(dy + Hr) * OW]
    h1 = jnp.dot(w1_ref[...], p1[...], preferred_element_type=jnp.float32)
    h1 = jnp.maximum(h1 + b1_ref[...], 0.0)
    yy = lax.broadcasted_iota(jnp.int32, h1.shape, 1) // OW + (r0 - 1)
    h1f[...] = jnp.where((yy >= 0) & (yy < OH), h1, 0.0).astype(jnp.bfloat16)

    h1_taps = _rolls(h1f[...], OW)
    for dy in range(3):
        for dx in range(3):
            k0 = (dy * 3 + dx) * Cout
            p2[k0:k0 + Cout, :] = h1_taps[dx][:, dy * OW:(dy + TH) * OW]
    h2 = jnp.dot(w2_ref[...], p2[...], preferred_element_type=jnp.float32)
    o_ref[...] = jnp.maximum(h2 + b2_ref[...], 0.0)[None]


def kernel(x1, x2, w_up, b_up, w1, b1, w2, b2):
    N, Cin, H, W = x1.shape
    Chalf = Cin // 2
    OH, OW = 2 * H, 2 * W
    Cout = w1.shape[0]
    TH = 32
    n_strips = OH // TH
    CW = TH // 2 + 4

    x1p = jnp.repeat(
        jnp.pad(x1.astype(jnp.bfloat16), ((0, 0), (0, 0), (2, 2), (0, 0))),
        2, axis=-1).reshape(N, Cin, (H + 4) * OW)
    x2p = jnp.pad(x2.astype(jnp.bfloat16), ((0, 0), (0, 0), (2, 2), (0, 0))
                  ).reshape(N, Chalf, (OH + 4) * OW)

    wupT = jnp.transpose(w_up, (2, 3, 1, 0)).reshape(4 * Chalf, Cin).astype(jnp.bfloat16)
    w1t = jnp.transpose(w1, (0, 2, 3, 1)).reshape(Cout, 9 * Cin).astype(jnp.bfloat16)
    w2t = jnp.transpose(w2, (0, 2, 3, 1)).reshape(Cout, 9 * Cout).astype(jnp.bfloat16)
    bupc = b_up.reshape(Chalf, 1)
    b1c = b1.reshape(Cout, 1)
    b2c = b2.reshape(Cout, 1)

    body = functools.partial(
        _fused_body, TH=TH, H=H, W=W, OH=OH, OW=OW,
        Chalf=Chalf, Cin=Cin, Cout=Cout)

    out = pl.pallas_call(
        body,
        out_shape=jax.ShapeDtypeStruct((N, Cout, OH * OW), jnp.float32),
        grid=(N, n_strips),
        in_specs=[
            pl.BlockSpec(memory_space=pl.ANY),
            pl.BlockSpec(memory_space=pl.ANY),
            pl.BlockSpec((4 * Chalf, Cin), lambda n, s: (0, 0)),
            pl.BlockSpec((Chalf, 1), lambda n, s: (0, 0)),
            pl.BlockSpec((Cout, 9 * Cin), lambda n, s: (0, 0)),
            pl.BlockSpec((Cout, 1), lambda n, s: (0, 0)),
            pl.BlockSpec((Cout, 9 * Cout), lambda n, s: (0, 0)),
            pl.BlockSpec((Cout, 1), lambda n, s: (0, 0)),
        ],
        out_specs=pl.BlockSpec((1, Cout, TH * OW), lambda n, s: (n, 0, s)),
        scratch_shapes=[
            pltpu.VMEM((Cin, CW * OW), jnp.bfloat16),
            pltpu.VMEM((Chalf, (TH + 4) * OW), jnp.bfloat16),
            pltpu.VMEM((Chalf, (2 * CW) * OW), jnp.bfloat16),
            pltpu.VMEM((9 * Cin, (TH + 2) * OW), jnp.bfloat16),
            pltpu.VMEM((Cout, (TH + 2) * OW), jnp.bfloat16),
            pltpu.VMEM((9 * Cout, TH * OW), jnp.bfloat16),
            pltpu.SemaphoreType.DMA((2,)),
        ],
        compiler_params=pltpu.CompilerParams(
            dimension_semantics=("parallel", "parallel")),
    )(x1p, x2p, wupT, bupc, w1t, b1c, w2t, b2c)

    return out.reshape(N, Cout, OH, OW)


# speedup vs baseline: 4.8492x; 1.0777x over previous
import functools

import jax
import jax.numpy as jnp
from jax import lax
from jax.experimental import pallas as pl
from jax.experimental.pallas import tpu as pltpu


def _rolls(arr, w):
    col = lax.broadcasted_iota(jnp.int32, arr.shape, 1) % w
    rot_l = jnp.concatenate([arr[:, -1:], arr[:, :-1]], axis=1)
    rot_r = jnp.concatenate([arr[:, 1:], arr[:, :1]], axis=1)
    left = jnp.where(col == 0, 0.0, rot_l)
    right = jnp.where(col == w - 1, 0.0, rot_r)
    return (left, arr, right)


def _fused_body(
    x1_hbm, x2_hbm, wup_ref, bup_ref, w1_ref, b1_ref, w2_ref, b2_ref,
    o_ref,
    xw, x2b, upf, p1, h1f, p2, sem,
    *, TH, H, W, OH, OW, Chalf, Cin, Cout,
):
    n = pl.program_id(0)
    s = pl.program_id(1)
    r0 = s * TH
    CW = TH // 2 + 4
    Hr = TH + 2

    cp1 = pltpu.make_async_copy(
        x1_hbm.at[n, :, pl.ds((TH // 2) * s * OW, CW * OW)], xw, sem.at[0])
    cp2 = pltpu.make_async_copy(
        x2_hbm.at[n, :, pl.ds(r0 * OW, (TH + 4) * OW)], x2b, sem.at[1])
    cp1.start()
    cp2.start()
    cp1.wait()
    cp2.wait()

    um = jnp.dot(wup_ref[...], xw[...], preferred_element_type=jnp.float32)
    lane = lax.broadcasted_iota(jnp.int32, (Chalf, CW * OW), 1)
    v_even = (lane % 2) == 0
    ev = jnp.where(v_even, um[0:Chalf], um[Chalf:2 * Chalf])
    od = jnp.where(v_even, um[2 * Chalf:3 * Chalf], um[3 * Chalf:4 * Chalf])
    u_ev = 2 * (TH // 2) * s - 4 + 2 * (lane // OW)
    bup = bup_ref[...]
    ev = jnp.where((u_ev >= 0) & (u_ev < OH), ev + bup, 0.0).astype(jnp.bfloat16)
    od = jnp.where((u_ev + 1 >= 0) & (u_ev + 1 < OH), od + bup, 0.0).astype(jnp.bfloat16)
    for i in range(CW):
        upf[:, (2 * i) * OW:(2 * i + 1) * OW] = ev[:, i * OW:(i + 1) * OW]
        upf[:, (2 * i + 1) * OW:(2 * i + 2) * OW] = od[:, i * OW:(i + 1) * OW]

    up_taps = _rolls(upf[...], OW)
    x2_taps = _rolls(x2b[...], OW)
    for dy in range(3):
        for dx in range(3):
            k0 = (dy * 3 + dx) * Cin
            p1[k0:k0 + Chalf, :] = up_taps[dx][:, (2 + dy) * OW:(2 + dy + Hr) * OW]
            p1[k0 + Chalf:k0 + Cin, :] = x2_taps[dx][:, dy * OW:(dy + Hr) * OW]
    h1 = jnp.dot(w1_ref[...], p1[...], preferred_element_type=jnp.float32)
    h1 = jnp.maximum(h1 + b1_ref[...], 0.0)
    yy = lax.broadcasted_iota(jnp.int32, h1.shape, 1) // OW + (r0 - 1)
    h1f[...] = jnp.where((yy >= 0) & (yy < OH), h1, 0.0).astype(jnp.bfloat16)

    h1_taps = _rolls(h1f[...], OW)
    for dy in range(3):
        for dx in range(3):
            k0 = (dy * 3 + dx) * Cout
            p2[k0:k0 + Cout, :] = h1_taps[dx][:, dy * OW:(dy + TH) * OW]
    h2 = jnp.dot(w2_ref[...], p2[...], preferred_element_type=jnp.float32)
    o_ref[...] = jnp.maximum(h2 + b2_ref[...], 0.0)[None]


def kernel(x1, x2, w_up, b_up, w1, b1, w2, b2):
    N, Cin, H, W = x1.shape
    Chalf = Cin // 2
    OH, OW = 2 * H, 2 * W
    Cout = w1.shape[0]
    TH = 64
    n_strips = OH // TH
    CW = TH // 2 + 4

    x1p = jnp.repeat(
        jnp.pad(x1.astype(jnp.bfloat16), ((0, 0), (0, 0), (2, 2), (0, 0))),
        2, axis=-1).reshape(N, Cin, (H + 4) * OW)
    x2p = jnp.pad(x2.astype(jnp.bfloat16), ((0, 0), (0, 0), (2, 2), (0, 0))
                  ).reshape(N, Chalf, (OH + 4) * OW)

    wupT = jnp.transpose(w_up, (2, 3, 1, 0)).reshape(4 * Chalf, Cin).astype(jnp.bfloat16)
    w1t = jnp.transpose(w1, (0, 2, 3, 1)).reshape(Cout, 9 * Cin).astype(jnp.bfloat16)
    w2t = jnp.transpose(w2, (0, 2, 3, 1)).reshape(Cout, 9 * Cout).astype(jnp.bfloat16)
    bupc = b_up.reshape(Chalf, 1)
    b1c = b1.reshape(Cout, 1)
    b2c = b2.reshape(Cout, 1)

    body = functools.partial(
        _fused_body, TH=TH, H=H, W=W, OH=OH, OW=OW,
        Chalf=Chalf, Cin=Cin, Cout=Cout)

    out = pl.pallas_call(
        body,
        out_shape=jax.ShapeDtypeStruct((N, Cout, OH * OW), jnp.float32),
        grid=(N, n_strips),
        in_specs=[
            pl.BlockSpec(memory_space=pl.ANY),
            pl.BlockSpec(memory_space=pl.ANY),
            pl.BlockSpec((4 * Chalf, Cin), lambda n, s: (0, 0)),
            pl.BlockSpec((Chalf, 1), lambda n, s: (0, 0)),
            pl.BlockSpec((Cout, 9 * Cin), lambda n, s: (0, 0)),
            pl.BlockSpec((Cout, 1), lambda n, s: (0, 0)),
            pl.BlockSpec((Cout, 9 * Cout), lambda n, s: (0, 0)),
            pl.BlockSpec((Cout, 1), lambda n, s: (0, 0)),
        ],
        out_specs=pl.BlockSpec((1, Cout, TH * OW), lambda n, s: (n, 0, s)),
        scratch_shapes=[
            pltpu.VMEM((Cin, CW * OW), jnp.bfloat16),
            pltpu.VMEM((Chalf, (TH + 4) * OW), jnp.bfloat16),
            pltpu.VMEM((Chalf, (2 * CW) * OW), jnp.bfloat16),
            pltpu.VMEM((9 * Cin, (TH + 2) * OW), jnp.bfloat16),
            pltpu.VMEM((Cout, (TH + 2) * OW), jnp.bfloat16),
            pltpu.VMEM((9 * Cout, TH * OW), jnp.bfloat16),
            pltpu.SemaphoreType.DMA((2,)),
        ],
        compiler_params=pltpu.CompilerParams(
            dimension_semantics=("parallel", "parallel")),
    )(x1p, x2p, wupT, bupc, w1t, b1c, w2t, b2c)

    return out.reshape(N, Cout, OH, OW)
